# baseline (speedup 1.0000x reference)
import functools

import jax
import jax.numpy as jnp
from jax import lax
from jax.experimental import pallas as pl
from jax.experimental.pallas import tpu as pltpu
from jax.experimental.pallas import tpu_sc as plsc

_LANES = 128


def _transpose_body(t_ref, o_ref):
    o_ref[:, 0:64] = jnp.transpose(t_ref[...], (1, 0))
    o_ref[:, 64:128] = jnp.zeros_like(o_ref[:, 64:128])


def _transpose_table(emb_t, col_block):
    d, vocab1 = emb_t.shape
    grid = pl.cdiv(vocab1, col_block)
    return pl.pallas_call(
        _transpose_body,
        grid=(grid,),
        in_specs=[pl.BlockSpec((d, col_block), lambda i: (0, i))],
        out_specs=pl.BlockSpec((col_block, _LANES), lambda i: (i, 0)),
        out_shape=jax.ShapeDtypeStruct((grid * col_block, _LANES), jnp.float32),
    )(emb_t)


def _sc_gather(table, flat_idx, n_rows):
    try:
        info = plsc.get_sparse_core_info()
        nc, ns = info.num_cores, info.num_subcores
    except Exception:
        nc, ns = 2, 16
    nw = nc * ns
    assert n_rows % (8 * nw) == 0
    b_per_w = n_rows // nw
    mesh = plsc.VectorSubcoreMesh(core_axis_name="c", subcore_axis_name="s")

    @functools.partial(
        pl.kernel,
        mesh=mesh,
        compiler_params=pltpu.CompilerParams(use_tc_tiling_on_sc=False),
        out_type=jax.ShapeDtypeStruct((n_rows, _LANES), jnp.float32),
        scratch_types=[
            pltpu.VMEM((b_per_w,), jnp.int32),
            pltpu.VMEM((b_per_w, _LANES), jnp.float32),
            pltpu.SemaphoreType.DMA,
        ],
    )
    def gather_kernel(table_hbm, idx_hbm, out_hbm, idx_v, rows_v, sem):
        wid = lax.axis_index("s") * nc + lax.axis_index("c")
        base = wid * b_per_w
        pltpu.sync_copy(idx_hbm.at[pl.ds(base, b_per_w)], idx_v)
        pltpu.async_copy(table_hbm.at[idx_v], rows_v, sem).wait()
        pltpu.sync_copy(rows_v, out_hbm.at[pl.ds(base, b_per_w)])

    return gather_kernel(table, flat_idx)


def _pool_body(g_ref, o_ref, *, ctx, d):
    o_ref[...] = jnp.sum(g_ref[...], axis=0)[:, 0:d] * (1.0 / ctx)


def _pool(gathered3, d):
    ctx, batch, lanes = gathered3.shape
    return pl.pallas_call(
        functools.partial(_pool_body, ctx=ctx, d=d),
        out_shape=jax.ShapeDtypeStruct((batch, d), jnp.float32),
    )(gathered3)


def _matmul_body(wt_ref, p_ref, o_ref):
    o_ref[...] = lax.dot_general(
        wt_ref[...],
        p_ref[...],
        dimension_numbers=(((0,), (1,)), ((), ())),
        preferred_element_type=jnp.float32,
        precision=lax.Precision.DEFAULT,
    )


def _matmul_t(w_t, pooled, row_block):
    d, vocab = w_t.shape
    batch = pooled.shape[0]
    grid = pl.cdiv(vocab, row_block)
    return pl.pallas_call(
        _matmul_body,
        grid=(grid,),
        in_specs=[
            pl.BlockSpec((d, row_block), lambda i: (0, i)),
            pl.BlockSpec((batch, d), lambda i: (0, 0)),
        ],
        out_specs=pl.BlockSpec((row_block, batch), lambda i: (i, 0)),
        out_shape=jax.ShapeDtypeStruct((vocab, batch), jnp.float32),
        compiler_params=pltpu.CompilerParams(
            vmem_limit_bytes=128 * 1024 * 1024,
        ),
    )(w_t, pooled)


def kernel(x, emb_table, W_out):
    batch, ctx = x.shape
    vocab, d = W_out.shape
    flat_idx = x.astype(jnp.int32).T.reshape(-1)
    table = _transpose_table(emb_table.T, col_block=2048)
    gathered = _sc_gather(table, flat_idx, batch * ctx)
    gathered3 = gathered.reshape(ctx, batch, _LANES)
    pooled = _pool(gathered3, d)
    logits_t = _matmul_t(W_out.T, pooled, row_block=6144)
    return logits_t.T


# speedup vs baseline: 1.0090x; 1.0090x over previous
import functools

import jax
import jax.numpy as jnp
from jax import lax
from jax.experimental import pallas as pl
from jax.experimental.pallas import tpu as pltpu
from jax.experimental.pallas import tpu_sc as plsc

_LANES = 128


def _transpose_body(t_ref, o_ref):
    o_ref[:, 0:64] = jnp.transpose(t_ref[...], (1, 0))
    o_ref[:, 64:128] = jnp.zeros_like(o_ref[:, 64:128])


def _transpose_table(emb_t, col_block):
    d, vocab1 = emb_t.shape
    grid = pl.cdiv(vocab1, col_block)
    return pl.pallas_call(
        _transpose_body,
        grid=(grid,),
        in_specs=[pl.BlockSpec((d, col_block), lambda i: (0, i))],
        out_specs=pl.BlockSpec((col_block, _LANES), lambda i: (i, 0)),
        out_shape=jax.ShapeDtypeStruct((grid * col_block, _LANES), jnp.float32),
    )(emb_t)


def _sc_gather(table, flat_idx, n_rows):
    try:
        info = plsc.get_sparse_core_info()
        nc, ns = info.num_cores, info.num_subcores
    except Exception:
        nc, ns = 2, 16
    nw = nc * ns
    assert n_rows % (8 * nw) == 0
    b_per_w = n_rows // nw
    mesh = plsc.VectorSubcoreMesh(core_axis_name="c", subcore_axis_name="s")

    @functools.partial(
        pl.kernel,
        mesh=mesh,
        compiler_params=pltpu.CompilerParams(use_tc_tiling_on_sc=False),
        out_type=jax.ShapeDtypeStruct((n_rows, _LANES), jnp.float32),
        scratch_types=[
            pltpu.VMEM((b_per_w,), jnp.int32),
            pltpu.VMEM((b_per_w, _LANES), jnp.float32),
            pltpu.SemaphoreType.DMA,
        ],
    )
    def gather_kernel(table_hbm, idx_hbm, out_hbm, idx_v, rows_v, sem):
        wid = lax.axis_index("s") * nc + lax.axis_index("c")
        base = wid * b_per_w
        pltpu.sync_copy(idx_hbm.at[pl.ds(base, b_per_w)], idx_v)
        pltpu.async_copy(table_hbm.at[idx_v], rows_v, sem).wait()
        pltpu.sync_copy(rows_v, out_hbm.at[pl.ds(base, b_per_w)])

    return gather_kernel(table, flat_idx)


def _fused_body(wt_ref, g_ref, o_ref, pooled_ref, *, ctx, batch, d):
    @pl.when(pl.program_id(0) == 0)
    def _pool():
        g3 = g_ref[...].reshape(ctx, batch, _LANES)
        pooled_ref[...] = jnp.sum(g3, axis=0)[:, 0:d] * (1.0 / ctx)

    o_ref[...] = lax.dot_general(
        wt_ref[...],
        pooled_ref[...],
        dimension_numbers=(((0,), (1,)), ((), ())),
        preferred_element_type=jnp.float32,
        precision=lax.Precision.DEFAULT,
    )


def _pool_matmul(w_t, gathered, ctx, row_block):
    d, vocab = w_t.shape
    batch = gathered.shape[0] // ctx
    grid = pl.cdiv(vocab, row_block)
    return pl.pallas_call(
        functools.partial(_fused_body, ctx=ctx, batch=batch, d=d),
        grid=(grid,),
        in_specs=[
            pl.BlockSpec((d, row_block), lambda i: (0, i)),
            pl.BlockSpec(gathered.shape, lambda i: (0, 0)),
        ],
        out_specs=pl.BlockSpec((row_block, batch), lambda i: (i, 0)),
        out_shape=jax.ShapeDtypeStruct((vocab, batch), jnp.float32),
        scratch_shapes=[pltpu.VMEM((batch, d), jnp.float32)],
    )(w_t, gathered)


def kernel(x, emb_table, W_out):
    batch, ctx = x.shape
    vocab, d = W_out.shape
    flat_idx = x.astype(jnp.int32).T.reshape(-1)
    table = _transpose_table(emb_table.T, col_block=2048)
    gathered = _sc_gather(table, flat_idx, batch * ctx)
    logits_t = _pool_matmul(W_out.T, gathered, ctx, row_block=4096)
    return logits_t.T


# speedup vs baseline: 1.0688x; 1.0593x over previous
import functools

import jax
import jax.numpy as jnp
from jax import lax
from jax.experimental import pallas as pl
from jax.experimental.pallas import tpu as pltpu
from jax.experimental.pallas import tpu_sc as plsc

_LANES = 128


def _transpose_body(t_ref, o_ref):
    o_ref[:, 0:64] = jnp.transpose(t_ref[...], (1, 0))
    o_ref[:, 64:128] = jnp.zeros_like(o_ref[:, 64:128])


def _transpose_table(emb_t, col_block):
    d, vocab1 = emb_t.shape
    grid = pl.cdiv(vocab1, col_block)
    return pl.pallas_call(
        _transpose_body,
        grid=(grid,),
        in_specs=[pl.BlockSpec((d, col_block), lambda i: (0, i))],
        out_specs=pl.BlockSpec((col_block, _LANES), lambda i: (i, 0)),
        out_shape=jax.ShapeDtypeStruct((grid * col_block, _LANES), jnp.float32),
    )(emb_t)


def _sc_gather(table, flat_idx, n_rows):
    try:
        info = plsc.get_sparse_core_info()
        nc, ns = info.num_cores, info.num_subcores
    except Exception:
        nc, ns = 2, 16
    nw = nc * ns
    assert n_rows % (8 * nw) == 0
    b_per_w = n_rows // nw
    mesh = plsc.VectorSubcoreMesh(core_axis_name="c", subcore_axis_name="s")

    @functools.partial(
        pl.kernel,
        mesh=mesh,
        compiler_params=pltpu.CompilerParams(use_tc_tiling_on_sc=False),
        out_type=jax.ShapeDtypeStruct((n_rows, _LANES), jnp.float32),
        scratch_types=[
            pltpu.VMEM((b_per_w,), jnp.int32),
            pltpu.VMEM((b_per_w, _LANES), jnp.float32),
            pltpu.SemaphoreType.DMA,
        ],
    )
    def gather_kernel(table_hbm, idx_hbm, out_hbm, idx_v, rows_v, sem):
        wid = lax.axis_index("s") * nc + lax.axis_index("c")
        base = wid * b_per_w
        pltpu.sync_copy(idx_hbm.at[pl.ds(base, b_per_w)], idx_v)
        pltpu.async_copy(table_hbm.at[idx_v], rows_v, sem).wait()
        pltpu.sync_copy(rows_v, out_hbm.at[pl.ds(base, b_per_w)])

    return gather_kernel(table, flat_idx)


def _fused_body(wt_ref, g_ref, o_ref, pooled_ref, *, ctx, batch, d):
    @pl.when(pl.program_id(0) == 0)
    def _pool():
        g3 = g_ref[...].reshape(ctx, batch, _LANES)
        pooled_ref[...] = jnp.sum(g3, axis=0)[:, 0:d] * (1.0 / ctx)

    o_ref[...] = lax.dot_general(
        wt_ref[...],
        pooled_ref[...],
        dimension_numbers=(((0,), (1,)), ((), ())),
        preferred_element_type=jnp.float32,
        precision=lax.Precision.DEFAULT,
    )


def _pool_matmul(w_t, gathered, ctx, row_block):
    d, vocab = w_t.shape
    batch = gathered.shape[0] // ctx
    grid = pl.cdiv(vocab, row_block)
    return pl.pallas_call(
        functools.partial(_fused_body, ctx=ctx, batch=batch, d=d),
        grid=(grid,),
        in_specs=[
            pl.BlockSpec((d, row_block), lambda i: (0, i)),
            pl.BlockSpec(gathered.shape, lambda i: (0, 0)),
        ],
        out_specs=pl.BlockSpec((row_block, batch), lambda i: (i, 0)),
        out_shape=jax.ShapeDtypeStruct((vocab, batch), jnp.float32),
        scratch_shapes=[pltpu.VMEM((batch, d), jnp.float32)],
    )(w_t, gathered)


def kernel(x, emb_table, W_out):
    batch, ctx = x.shape
    vocab, d = W_out.shape
    flat_idx = x.astype(jnp.int32).T.reshape(-1)
    table = _transpose_table(emb_table.T, col_block=4096)
    gathered = _sc_gather(table, flat_idx, batch * ctx)
    logits_t = _pool_matmul(W_out.T, gathered, ctx, row_block=4096)
    return logits_t.T


# speedup vs baseline: 1.1039x; 1.0328x over previous
import functools

import jax
import jax.numpy as jnp
from jax import lax
from jax.experimental import pallas as pl
from jax.experimental.pallas import tpu as pltpu
from jax.experimental.pallas import tpu_sc as plsc

_LANES = 128


def _transpose_body(t_ref, o_ref):
    o_ref[:, 0:64] = jnp.transpose(t_ref[...], (1, 0))
    o_ref[:, 64:128] = jnp.zeros_like(o_ref[:, 64:128])


def _transpose_table(emb_t, col_block):
    d, vocab1 = emb_t.shape
    grid = pl.cdiv(vocab1, col_block)
    return pl.pallas_call(
        _transpose_body,
        grid=(grid,),
        in_specs=[pl.BlockSpec((d, col_block), lambda i: (0, i))],
        out_specs=pl.BlockSpec((col_block, _LANES), lambda i: (i, 0)),
        out_shape=jax.ShapeDtypeStruct((grid * col_block, _LANES), jnp.float32),
    )(emb_t)


def _sc_gather(table, flat_idx, n_rows):
    try:
        info = plsc.get_sparse_core_info()
        nc, ns = info.num_cores, info.num_subcores
    except Exception:
        nc, ns = 2, 16
    nw = nc * ns
    assert n_rows % (8 * nw) == 0
    b_per_w = n_rows // nw
    mesh = plsc.VectorSubcoreMesh(core_axis_name="c", subcore_axis_name="s")

    @functools.partial(
        pl.kernel,
        mesh=mesh,
        compiler_params=pltpu.CompilerParams(use_tc_tiling_on_sc=False),
        out_type=jax.ShapeDtypeStruct((n_rows, _LANES), jnp.float32),
        scratch_types=[
            pltpu.VMEM((b_per_w,), jnp.int32),
            pltpu.VMEM((b_per_w, _LANES), jnp.float32),
            pltpu.SemaphoreType.DMA,
        ],
    )
    def gather_kernel(table_hbm, idx_hbm, out_hbm, idx_v, rows_v, sem):
        wid = lax.axis_index("s") * nc + lax.axis_index("c")
        base = wid * b_per_w
        pltpu.sync_copy(idx_hbm.at[pl.ds(base, b_per_w)], idx_v)
        pltpu.async_copy(table_hbm.at[idx_v], rows_v, sem).wait()
        pltpu.sync_copy(rows_v, out_hbm.at[pl.ds(base, b_per_w)])

    return gather_kernel(table, flat_idx)


def _fused_body(wt_ref, g_ref, o_ref, pooled_ref, *, ctx, batch, d):
    @pl.when(pl.program_id(0) == 0)
    def _pool():
        g3 = g_ref[...].reshape(ctx, batch, _LANES)
        pooled_ref[...] = jnp.sum(g3, axis=0)[:, 0:d] * (1.0 / ctx)

    o_ref[...] = lax.dot_general(
        wt_ref[...],
        pooled_ref[...],
        dimension_numbers=(((0,), (1,)), ((), ())),
        preferred_element_type=jnp.float32,
        precision=lax.Precision.DEFAULT,
    )


def _pool_matmul(w_t, gathered, ctx, row_block):
    d, vocab = w_t.shape
    batch = gathered.shape[0] // ctx
    grid = pl.cdiv(vocab, row_block)
    return pl.pallas_call(
        functools.partial(_fused_body, ctx=ctx, batch=batch, d=d),
        grid=(grid,),
        in_specs=[
            pl.BlockSpec((d, row_block), lambda i: (0, i)),
            pl.BlockSpec(gathered.shape, lambda i: (0, 0)),
        ],
        out_specs=pl.BlockSpec((row_block, batch), lambda i: (i, 0)),
        out_shape=jax.ShapeDtypeStruct((vocab, batch), jnp.float32),
        scratch_shapes=[pltpu.VMEM((batch, d), jnp.float32)],
    )(w_t, gathered)


def kernel(x, emb_table, W_out):
    batch, ctx = x.shape
    vocab, d = W_out.shape
    flat_idx = x.astype(jnp.int32).T.reshape(-1)
    table = _transpose_table(emb_table.T, col_block=8192)
    gathered = _sc_gather(table, flat_idx, batch * ctx)
    logits_t = _pool_matmul(W_out.T, gathered, ctx, row_block=4096)
    return logits_t.T


# speedup vs baseline: 1.1100x; 1.0056x over previous
import functools

import jax
import jax.numpy as jnp
from jax import lax
from jax.experimental import pallas as pl
from jax.experimental.pallas import tpu as pltpu
from jax.experimental.pallas import tpu_sc as plsc

_LANES = 128


def _transpose_body(t_ref, o_ref):
    o_ref[:, 0:64] = jnp.transpose(t_ref[...], (1, 0))
    o_ref[:, 64:128] = jnp.zeros_like(o_ref[:, 64:128])


def _transpose_table(emb_t, col_block):
    d, vocab1 = emb_t.shape
    grid = pl.cdiv(vocab1, col_block)
    return pl.pallas_call(
        _transpose_body,
        grid=(grid,),
        in_specs=[pl.BlockSpec((d, col_block), lambda i: (0, i))],
        out_specs=pl.BlockSpec((col_block, _LANES), lambda i: (i, 0)),
        out_shape=jax.ShapeDtypeStruct((grid * col_block, _LANES), jnp.float32),
    )(emb_t)


def _sc_gather(table, flat_idx, n_rows):
    try:
        info = plsc.get_sparse_core_info()
        nc, ns = info.num_cores, info.num_subcores
    except Exception:
        nc, ns = 2, 16
    nw = nc * ns
    assert n_rows % (8 * nw) == 0
    b_per_w = n_rows // nw
    mesh = plsc.VectorSubcoreMesh(core_axis_name="c", subcore_axis_name="s")

    @functools.partial(
        pl.kernel,
        mesh=mesh,
        compiler_params=pltpu.CompilerParams(use_tc_tiling_on_sc=False),
        out_type=jax.ShapeDtypeStruct((n_rows, _LANES), jnp.float32),
        scratch_types=[
            pltpu.VMEM((b_per_w,), jnp.int32),
            pltpu.VMEM((b_per_w, _LANES), jnp.float32),
            pltpu.SemaphoreType.DMA,
        ],
    )
    def gather_kernel(table_hbm, idx_hbm, out_hbm, idx_v, rows_v, sem):
        wid = lax.axis_index("s") * nc + lax.axis_index("c")
        base = wid * b_per_w
        pltpu.sync_copy(idx_hbm.at[pl.ds(base, b_per_w)], idx_v)
        pltpu.async_copy(table_hbm.at[idx_v], rows_v, sem).wait()
        pltpu.sync_copy(rows_v, out_hbm.at[pl.ds(base, b_per_w)])

    return gather_kernel(table, flat_idx)


def _fused_body(wt_ref, g_ref, o_ref, pooled_ref, *, ctx, batch, d):
    @pl.when(pl.program_id(0) == 0)
    def _pool():
        g3 = g_ref[...].reshape(ctx, batch, _LANES)
        pooled_ref[...] = jnp.sum(g3, axis=0)[:, 0:d] * (1.0 / ctx)

    o_ref[...] = lax.dot_general(
        wt_ref[...],
        pooled_ref[...],
        dimension_numbers=(((0,), (1,)), ((), ())),
        preferred_element_type=jnp.float32,
        precision=lax.Precision.DEFAULT,
    )


def _pool_matmul(w_t, gathered, ctx, row_block):
    d, vocab = w_t.shape
    batch = gathered.shape[0] // ctx
    grid = pl.cdiv(vocab, row_block)
    return pl.pallas_call(
        functools.partial(_fused_body, ctx=ctx, batch=batch, d=d),
        grid=(grid,),
        in_specs=[
            pl.BlockSpec((d, row_block), lambda i: (0, i)),
            pl.BlockSpec(gathered.shape, lambda i: (0, 0)),
        ],
        out_specs=pl.BlockSpec((row_block, batch), lambda i: (i, 0)),
        out_shape=jax.ShapeDtypeStruct((vocab, batch), jnp.float32),
        scratch_shapes=[pltpu.VMEM((batch, d), jnp.float32)],
    )(w_t, gathered)


def kernel(x, emb_table, W_out):
    batch, ctx = x.shape
    vocab, d = W_out.shape
    flat_idx = x.astype(jnp.int32).T.reshape(-1)
    table = _transpose_table(emb_table.T, col_block=16384)
    gathered = _sc_gather(table, flat_idx, batch * ctx)
    logits_t = _pool_matmul(W_out.T, gathered, ctx, row_block=4096)
    return logits_t.T


# speedup vs baseline: 1.1164x; 1.0057x over previous
import functools

import jax
import jax.numpy as jnp
from jax import lax
from jax.experimental import pallas as pl
from jax.experimental.pallas import tpu as pltpu
from jax.experimental.pallas import tpu_sc as plsc

_LANES = 128


def _transpose_body(t_ref, o_ref):
    o_ref[:, 0:64] = jnp.transpose(t_ref[...], (1, 0))
    o_ref[:, 64:128] = jnp.zeros_like(o_ref[:, 64:128])


def _transpose_table(emb_t, col_block):
    d, vocab1 = emb_t.shape
    grid = pl.cdiv(vocab1, col_block)
    return pl.pallas_call(
        _transpose_body,
        grid=(grid,),
        in_specs=[pl.BlockSpec((d, col_block), lambda i: (0, i))],
        out_specs=pl.BlockSpec((col_block, _LANES), lambda i: (i, 0)),
        out_shape=jax.ShapeDtypeStruct((grid * col_block, _LANES), jnp.float32),
    )(emb_t)


def _sc_gather(table, flat_idx, n_rows):
    try:
        info = plsc.get_sparse_core_info()
        nc, ns = info.num_cores, info.num_subcores
    except Exception:
        nc, ns = 2, 16
    nw = nc * ns
    assert n_rows % (8 * nw) == 0
    b_per_w = n_rows // nw
    mesh = plsc.VectorSubcoreMesh(core_axis_name="c", subcore_axis_name="s")

    @functools.partial(
        pl.kernel,
        mesh=mesh,
        compiler_params=pltpu.CompilerParams(use_tc_tiling_on_sc=False),
        out_type=jax.ShapeDtypeStruct((n_rows, _LANES), jnp.float32),
        scratch_types=[
            pltpu.VMEM((b_per_w,), jnp.int32),
            pltpu.VMEM((b_per_w, _LANES), jnp.float32),
            pltpu.SemaphoreType.DMA,
        ],
    )
    def gather_kernel(table_hbm, idx_hbm, out_hbm, idx_v, rows_v, sem):
        wid = lax.axis_index("s") * nc + lax.axis_index("c")
        base = wid * b_per_w
        pltpu.sync_copy(idx_hbm.at[pl.ds(base, b_per_w)], idx_v)
        pltpu.async_copy(table_hbm.at[idx_v], rows_v, sem).wait()
        pltpu.sync_copy(rows_v, out_hbm.at[pl.ds(base, b_per_w)])

    return gather_kernel(table, flat_idx)


def _fused_body(wt_ref, g_ref, o_ref, pooled_ref, *, ctx, batch, d):
    @pl.when(pl.program_id(0) == 0)
    def _pool():
        g3 = g_ref[...].reshape(ctx, batch, _LANES)
        pooled_ref[...] = jnp.sum(g3, axis=0)[:, 0:d] * (1.0 / ctx)

    o_ref[...] = lax.dot_general(
        wt_ref[...],
        pooled_ref[...],
        dimension_numbers=(((0,), (1,)), ((), ())),
        preferred_element_type=jnp.float32,
        precision=lax.Precision.DEFAULT,
    )


def _pool_matmul(w_t, gathered, ctx, row_block):
    d, vocab = w_t.shape
    batch = gathered.shape[0] // ctx
    grid = pl.cdiv(vocab, row_block)
    return pl.pallas_call(
        functools.partial(_fused_body, ctx=ctx, batch=batch, d=d),
        grid=(grid,),
        in_specs=[
            pl.BlockSpec((d, row_block), lambda i: (0, i)),
            pl.BlockSpec(gathered.shape, lambda i: (0, 0)),
        ],
        out_specs=pl.BlockSpec((row_block, batch), lambda i: (i, 0)),
        out_shape=jax.ShapeDtypeStruct((vocab, batch), jnp.float32),
        scratch_shapes=[pltpu.VMEM((batch, d), jnp.float32)],
    )(w_t, gathered)


def kernel(x, emb_table, W_out):
    batch, ctx = x.shape
    vocab, d = W_out.shape
    flat_idx = x.astype(jnp.int32).T.reshape(-1)
    table = _transpose_table(emb_table.T, col_block=12800)
    gathered = _sc_gather(table, flat_idx, batch * ctx)
    logits_t = _pool_matmul(W_out.T, gathered, ctx, row_block=4096)
    return logits_t.T


# speedup vs baseline: 1.1165x; 1.0001x over previous
import functools

import jax
import jax.numpy as jnp
from jax import lax
from jax.experimental import pallas as pl
from jax.experimental.pallas import tpu as pltpu
from jax.experimental.pallas import tpu_sc as plsc

_LANES = 128


def _transpose_body(t_ref, o_ref):
    o_ref[:, 0:64] = jnp.transpose(t_ref[...], (1, 0))
    o_ref[:, 64:128] = jnp.zeros_like(o_ref[:, 64:128])


def _transpose_table(emb_t, col_block):
    d, vocab1 = emb_t.shape
    grid = pl.cdiv(vocab1, col_block)
    return pl.pallas_call(
        _transpose_body,
        grid=(grid,),
        in_specs=[pl.BlockSpec((d, col_block), lambda i: (0, i))],
        out_specs=pl.BlockSpec((col_block, _LANES), lambda i: (i, 0)),
        out_shape=jax.ShapeDtypeStruct((grid * col_block, _LANES), jnp.float32),
    )(emb_t)


def _sc_gather(table, flat_idx, n_rows):
    try:
        info = plsc.get_sparse_core_info()
        nc, ns = info.num_cores, info.num_subcores
    except Exception:
        nc, ns = 2, 16
    nw = nc * ns
    assert n_rows % (8 * nw) == 0
    b_per_w = n_rows // nw
    mesh = plsc.VectorSubcoreMesh(core_axis_name="c", subcore_axis_name="s")

    @functools.partial(
        pl.kernel,
        mesh=mesh,
        compiler_params=pltpu.CompilerParams(use_tc_tiling_on_sc=False),
        out_type=jax.ShapeDtypeStruct((n_rows, _LANES), jnp.float32),
        scratch_types=[
            pltpu.VMEM((b_per_w,), jnp.int32),
            pltpu.VMEM((b_per_w, _LANES), jnp.float32),
            pltpu.SemaphoreType.DMA,
        ],
    )
    def gather_kernel(table_hbm, idx_hbm, out_hbm, idx_v, rows_v, sem):
        wid = lax.axis_index("s") * nc + lax.axis_index("c")
        base = wid * b_per_w
        pltpu.sync_copy(idx_hbm.at[pl.ds(base, b_per_w)], idx_v)
        pltpu.async_copy(table_hbm.at[idx_v], rows_v, sem).wait()
        pltpu.sync_copy(rows_v, out_hbm.at[pl.ds(base, b_per_w)])

    return gather_kernel(table, flat_idx)


def _fused_body(wt_ref, g_ref, o_ref, pooled_ref, *, ctx, batch, d):
    @pl.when(pl.program_id(0) == 0)
    def _pool():
        g3 = g_ref[...].reshape(ctx, batch, _LANES)
        pooled_ref[...] = jnp.sum(g3, axis=0)[:, 0:d] * (1.0 / ctx)

    o_ref[...] = lax.dot_general(
        wt_ref[...],
        pooled_ref[...],
        dimension_numbers=(((0,), (1,)), ((), ())),
        preferred_element_type=jnp.float32,
        precision=lax.Precision.DEFAULT,
    )


def _pool_matmul(w_t, gathered, ctx, row_block):
    d, vocab = w_t.shape
    batch = gathered.shape[0] // ctx
    grid = pl.cdiv(vocab, row_block)
    return pl.pallas_call(
        functools.partial(_fused_body, ctx=ctx, batch=batch, d=d),
        grid=(grid,),
        in_specs=[
            pl.BlockSpec((d, row_block), lambda i: (0, i)),
            pl.BlockSpec(gathered.shape, lambda i: (0, 0)),
        ],
        out_specs=pl.BlockSpec((row_block, batch), lambda i: (i, 0)),
        out_shape=jax.ShapeDtypeStruct((vocab, batch), jnp.float32),
        scratch_shapes=[pltpu.VMEM((batch, d), jnp.float32)],
    )(w_t, gathered)


def kernel(x, emb_table, W_out):
    batch, ctx = x.shape
    vocab, d = W_out.shape
    flat_idx = x.astype(jnp.int32).T.reshape(-1)
    table = _transpose_table(emb_table.T, col_block=12800)
    gathered = _sc_gather(table, flat_idx, batch * ctx)
    logits_t = _pool_matmul(W_out.T, gathered, ctx, row_block=5120)
    return logits_t.T
